# baseline (speedup 1.0000x reference)
import jax
import jax.numpy as jnp
from jax.experimental import pallas as pl
from jax.experimental.pallas import tpu as pltpu


def _pair_kernel(x_ref, w1cat_ref, b1_ref, w2_ref, b2_ref, out_ref,
                 at_scr, bvt_scr, w2bc_scr, s_scr):
    H, n = at_scr.shape

    tx = jnp.tanh(x_ref[0])
    ab = jnp.dot(tx, w1cat_ref[:], preferred_element_type=jnp.float32)
    at_scr[:] = (ab[:, :H] + b1_ref[:]).T.astype(jnp.bfloat16)
    bvt_scr[:] = ab[:, H:].T.astype(jnp.bfloat16)
    w2bc_scr[:] = jnp.broadcast_to(
        w2_ref[:].T, (H, 128)).astype(jnp.bfloat16)

    b2v = b2_ref[0, 0]
    w2b = w2bc_scr[:]
    for rb in range(n // 128):
        bt = bvt_scr[:, rb * 128:(rb + 1) * 128]
        for l in range(128):
            r = rb * 128 + l
            bcol = bt[:, l:l + 1]
            for jb in range(n // 128):
                atb = at_scr[:, jb * 128:(jb + 1) * 128]
                p = jnp.tanh(atb + bcol) * w2b
                h = H
                while h > 16:
                    h //= 2
                    p = p[:h] + p[h:]
                s_r = jnp.sum(p, axis=0, dtype=jnp.float32)
                s_scr[r:r + 1, jb * 128:(jb + 1) * 128] = s_r[None, :] + b2v

    sv = s_scr[:]
    out_ref[0] = sv + sv.T


def kernel(x, W1, b1, W2, b2):
    B, n, F = x.shape
    H = W1.shape[0]

    w1cat = jnp.concatenate([W1[:, :F].T, W1[:, F:].T], axis=1)
    b1r = b1.reshape(1, H)
    w2r = W2.reshape(1, H)
    b2r = b2.reshape(1, 1)

    return pl.pallas_call(
        _pair_kernel,
        grid=(B,),
        in_specs=[
            pl.BlockSpec((1, n, F), lambda b: (b, 0, 0)),
            pl.BlockSpec((F, 2 * H), lambda b: (0, 0)),
            pl.BlockSpec((1, H), lambda b: (0, 0)),
            pl.BlockSpec((1, H), lambda b: (0, 0)),
            pl.BlockSpec((1, 1), lambda b: (0, 0)),
        ],
        out_specs=pl.BlockSpec((1, n, n), lambda b: (b, 0, 0)),
        out_shape=jax.ShapeDtypeStruct((B, n, n), jnp.float32),
        scratch_shapes=[
            pltpu.VMEM((H, n), jnp.bfloat16),
            pltpu.VMEM((H, n), jnp.bfloat16),
            pltpu.VMEM((H, 128), jnp.bfloat16),
            pltpu.VMEM((n, n), jnp.float32),
        ],
        compiler_params=pltpu.CompilerParams(
            dimension_semantics=("parallel",),
        ),
    )(x, w1cat, b1r, w2r, b2r)


# speedup vs baseline: 1.0417x; 1.0417x over previous
import jax
import jax.numpy as jnp
from jax.experimental import pallas as pl
from jax.experimental.pallas import tpu as pltpu


def _pair_kernel(x_ref, w1cat_ref, b1_ref, w2_ref, b2_ref, out_ref,
                 at_scr, bv_scr, w2bc_scr, s_scr):
    H, n = at_scr.shape

    tx = jnp.tanh(x_ref[0])
    ab = jnp.dot(tx, w1cat_ref[:], preferred_element_type=jnp.float32)
    at_scr[:] = (ab[:, :H] + b1_ref[:]).T.astype(jnp.bfloat16)
    bv_scr[:] = ab[:, H:].astype(jnp.bfloat16)
    w2bc_scr[:] = jnp.broadcast_to(
        w2_ref[:].T, (H, 128)).astype(jnp.bfloat16)

    b2v = b2_ref[0, 0]
    w2b = w2bc_scr[:]
    for r in range(n):
        bcol = bv_scr[r][:, None]
        for jb in range(n // 128):
            atb = at_scr[:, jb * 128:(jb + 1) * 128]
            p = jnp.tanh(atb + bcol) * w2b
            h = H
            while h > 16:
                h //= 2
                p = p[:h] + p[h:]
            s_r = jnp.sum(p, axis=0, dtype=jnp.float32)
            s_scr[r:r + 1, jb * 128:(jb + 1) * 128] = s_r[None, :] + b2v

    sv = s_scr[:]
    out_ref[0] = sv + sv.T


def kernel(x, W1, b1, W2, b2):
    B, n, F = x.shape
    H = W1.shape[0]

    w1cat = jnp.concatenate([W1[:, :F].T, W1[:, F:].T], axis=1)
    b1r = b1.reshape(1, H)
    w2r = W2.reshape(1, H)
    b2r = b2.reshape(1, 1)

    return pl.pallas_call(
        _pair_kernel,
        grid=(B,),
        in_specs=[
            pl.BlockSpec((1, n, F), lambda b: (b, 0, 0)),
            pl.BlockSpec((F, 2 * H), lambda b: (0, 0)),
            pl.BlockSpec((1, H), lambda b: (0, 0)),
            pl.BlockSpec((1, H), lambda b: (0, 0)),
            pl.BlockSpec((1, 1), lambda b: (0, 0)),
        ],
        out_specs=pl.BlockSpec((1, n, n), lambda b: (b, 0, 0)),
        out_shape=jax.ShapeDtypeStruct((B, n, n), jnp.float32),
        scratch_shapes=[
            pltpu.VMEM((H, n), jnp.bfloat16),
            pltpu.VMEM((H, n), jnp.bfloat16),
            pltpu.VMEM((H, 128), jnp.bfloat16),
            pltpu.VMEM((n, n), jnp.float32),
        ],
        compiler_params=pltpu.CompilerParams(
            dimension_semantics=("parallel",),
        ),
    )(x, w1cat, b1r, w2r, b2r)


# speedup vs baseline: 1.7921x; 1.7203x over previous
import jax
import jax.numpy as jnp
from jax.experimental import pallas as pl
from jax.experimental.pallas import tpu as pltpu


def _pair_kernel(x_ref, w1cat_ref, b1_ref, w2_ref, b2_ref, out_ref,
                 at_scr, bv_scr, w2bc_scr, s_scr):
    t = pl.program_id(1)
    T = pl.num_programs(1)
    H, n = at_scr.shape
    R = n // T

    @pl.when(t == 0)
    def _init():
        tx = jnp.tanh(x_ref[0])
        ab = jnp.dot(tx, w1cat_ref[:], preferred_element_type=jnp.float32)
        at_scr[:] = (ab[:, :H] + b1_ref[:]).T.astype(jnp.bfloat16)
        bv_scr[:] = ab[:, H:].astype(jnp.bfloat16)
        w2bc_scr[:] = jnp.broadcast_to(
            w2_ref[:].T, (H, n)).astype(jnp.bfloat16)

    base = t * R
    bv = bv_scr[pl.ds(base, R), :]
    at = at_scr[:]
    w2bc = w2bc_scr[:]
    b2v = b2_ref[0, 0]
    for r in range(R):
        p = jnp.tanh(at + bv[r][:, None]) * w2bc
        h = H
        while h > 16:
            h //= 2
            p = p[:h] + p[h:]
        s_r = jnp.sum(p, axis=0, dtype=jnp.float32)
        s_scr[pl.ds(base + r, 1), :] = s_r[None, :] + b2v

    @pl.when(t == T - 1)
    def _finalize():
        sv = s_scr[:]
        out_ref[0] = sv + sv.T


def kernel(x, W1, b1, W2, b2):
    B, n, F = x.shape
    H = W1.shape[0]
    T = 1

    w1cat = jnp.concatenate([W1[:, :F].T, W1[:, F:].T], axis=1)
    b1r = b1.reshape(1, H)
    w2r = W2.reshape(1, H)
    b2r = b2.reshape(1, 1)

    return pl.pallas_call(
        _pair_kernel,
        grid=(B, T),
        in_specs=[
            pl.BlockSpec((1, n, F), lambda b, t: (b, 0, 0)),
            pl.BlockSpec((F, 2 * H), lambda b, t: (0, 0)),
            pl.BlockSpec((1, H), lambda b, t: (0, 0)),
            pl.BlockSpec((1, H), lambda b, t: (0, 0)),
            pl.BlockSpec((1, 1), lambda b, t: (0, 0)),
        ],
        out_specs=pl.BlockSpec((1, n, n), lambda b, t: (b, 0, 0)),
        out_shape=jax.ShapeDtypeStruct((B, n, n), jnp.float32),
        scratch_shapes=[
            pltpu.VMEM((H, n), jnp.bfloat16),
            pltpu.VMEM((n, H), jnp.bfloat16),
            pltpu.VMEM((H, n), jnp.bfloat16),
            pltpu.VMEM((n, n), jnp.float32),
        ],
        compiler_params=pltpu.CompilerParams(
            dimension_semantics=("parallel", "arbitrary"),
        ),
    )(x, w1cat, b1r, w2r, b2r)
